# baseline (speedup 1.0000x reference)
import functools

import jax
import jax.numpy as jnp
import numpy as np
from jax import lax
from jax.experimental import pallas as pl
from jax.experimental.pallas import tpu as pltpu
from jax.experimental.pallas import tpu_sc as plsc

BZ = 8
HID = 256
K = 8192
RES_LAST = 513
KEY_MAX = 0x3F000000
NC, NS, L = 2, 16, 16

LEVELS = [(65, 128), (129, 256), (257, 384), (513, 640)]


def _up_mats(rp, wpp, r, wp):
    m = np.zeros((r, rp), np.float32)
    i = np.arange(rp)
    m[2 * i, i] = 1.0
    i = np.arange(rp - 1)
    m[2 * i + 1, i] = 0.5
    m[2 * i + 1, i + 1] = 0.5
    mt = np.zeros((wpp, wp), np.float32)
    mt[:rp, :r] = m.T
    return jnp.asarray(m), jnp.asarray(mt)


def _mk_up_kernel(rp, wpp, r, wp):
    rq = -(-r // 4)

    def body(prev_ref, m_ref, mt_ref, occ_ref, keys_ref, stats_ref):
        prev = prev_ref[0]
        tmp = jnp.dot(m_ref[:, :], prev, preferred_element_type=jnp.float32)
        out = jnp.dot(tmp, mt_ref[:, :], preferred_element_type=jnp.float32)
        occ_ref[0] = out
        a = jnp.abs(out - 0.5)
        key = KEY_MAX - lax.bitcast_convert_type(a, jnp.int32)
        col = lax.broadcasted_iota(jnp.int32, (r, wp), 1)
        key = jnp.where(col < r, key, -1)
        keys_ref[0] = key

        def bs(_, lohi):
            lo, hi = lohi
            mid = lo + (hi - lo + 1) // 2
            cnt = jnp.sum((key >= mid).astype(jnp.int32))
            ge = cnt >= K
            return (jnp.where(ge, mid, lo), jnp.where(ge, hi, mid - 1))

        lo, _ = lax.fori_loop(
            0, 31, bs, (jnp.int32(0), jnp.int32(KEY_MAX + 1)))
        t = lo
        mgt = (key > t).astype(jnp.int32)
        meq = (key == t).astype(jnp.int32)
        cg = [jnp.sum(mgt[q * rq:min((q + 1) * rq, r)]) for q in range(4)]
        ce = [jnp.sum(meq[q * rq:min((q + 1) * rq, r)]) for q in range(4)]
        c_gt = cg[0] + cg[1] + cg[2] + cg[3]
        m_take = K - c_gt
        vals = [t, m_take] + cg + ce + [jnp.int32(0)] * 6
        ii = lax.broadcasted_iota(jnp.int32, (1, 16), 1)
        acc = jnp.zeros((1, 16), jnp.int32)
        for e, v in enumerate(vals):
            acc = jnp.where(ii == e, v, acc)
        stats_ref[0] = acc

    def call(occ_prev, m, mt):
        return pl.pallas_call(
            body,
            grid=(BZ,),
            in_specs=[
                pl.BlockSpec((1, rp, wpp), lambda b: (b, 0, 0)),
                pl.BlockSpec((r, rp), lambda b: (0, 0)),
                pl.BlockSpec((wpp, wp), lambda b: (0, 0)),
            ],
            out_specs=[
                pl.BlockSpec((1, r, wp), lambda b: (b, 0, 0)),
                pl.BlockSpec((1, r, wp), lambda b: (b, 0, 0)),
                pl.BlockSpec((1, 1, 16), lambda b: (b, 0, 0)),
            ],
            out_shape=[
                jax.ShapeDtypeStruct((BZ, r, wp), jnp.float32),
                jax.ShapeDtypeStruct((BZ, r, wp), jnp.int32),
                jax.ShapeDtypeStruct((BZ, 1, 16), jnp.int32),
            ],
        )(occ_prev, m, mt)

    return call


def _mk_mlp_kernel(r, stride):
    half_step = float(np.float32(np.float32(1.0 / RES_LAST) / 2))

    def body(idx_ref, feat_ref, w1_ref, wf_ref, b1_ref, w2_ref, b2_ref,
             w3_ref, b3_ref, out_ref):
        idx = idx_ref[0]
        px = (idx % r).astype(jnp.float32) * stride
        py = (idx // r).astype(jnp.float32) * stride
        cx = (px / 513.0 + half_step) * 2.0 - 1.0
        cy = (py / 513.0 + half_step) * 2.0 - 1.0
        fw = jnp.dot(feat_ref[0], wf_ref[:, :],
                     preferred_element_type=jnp.float32) + b1_ref[:, :]
        h1 = jnp.maximum(cx * w1_ref[0:1, :] + cy * w1_ref[1:2, :] + fw, 0.0)
        h2 = jnp.maximum(
            jnp.dot(h1, w2_ref[:, :], preferred_element_type=jnp.float32)
            + b2_ref[:, :], 0.0)
        z = jnp.sum(h2 * w3_ref[:, :], axis=1, keepdims=True) + b3_ref[:, :]
        out_ref[0] = jax.nn.sigmoid(z)

    TK = 2048

    def call(idx3, feat, w1, wf, b1r, w2, b2r, w3r, b3r):
        return pl.pallas_call(
            body,
            grid=(BZ, K // TK),
            in_specs=[
                pl.BlockSpec((1, TK, 1), lambda b, t: (b, t, 0)),
                pl.BlockSpec((1, 1, HID), lambda b, t: (b, 0, 0)),
                pl.BlockSpec((2, HID), lambda b, t: (0, 0)),
                pl.BlockSpec((HID, HID), lambda b, t: (0, 0)),
                pl.BlockSpec((1, HID), lambda b, t: (0, 0)),
                pl.BlockSpec((HID, HID), lambda b, t: (0, 0)),
                pl.BlockSpec((1, HID), lambda b, t: (0, 0)),
                pl.BlockSpec((1, HID), lambda b, t: (0, 0)),
                pl.BlockSpec((1, 1), lambda b, t: (0, 0)),
            ],
            out_specs=pl.BlockSpec((1, TK, 1), lambda b, t: (b, t, 0)),
            out_shape=jax.ShapeDtypeStruct((BZ, K, 1), jnp.float32),
        )(idx3, feat.reshape(BZ, 1, HID), w1, wf, b1r, w2, b2r, w3r, b3r)

    return call


def _mk_compact_kernel(r, wp):
    rq = -(-r // 4)
    tot = r * wp
    mesh = plsc.VectorSubcoreMesh(core_axis_name="c", subcore_axis_name="s")

    def body(keys_hbm, stats_hbm, out_hbm, kbuf, gtbuf, eqbuf, statv,
             offs_ref, sem):
        cid = lax.axis_index("c")
        sid = lax.axis_index("s")
        wid = cid * NS + sid
        b = lax.div(wid, 4)
        q = lax.rem(wid, 4)
        lane = lax.iota(jnp.int32, 16)

        pltpu.sync_copy(stats_hbm.at[b], statv)
        sv = statv[...]

        def ssum(msk):
            return jnp.sum(jnp.where(msk, sv, jnp.zeros_like(sv)))

        t = ssum(lane == 0)
        m_take = ssum(lane == 1)
        cgt_q = ssum(lane == 2 + q)
        gt_before = ssum((lane >= 2) & (lane < 2 + q))
        c_gt_tot = ssum((lane >= 2) & (lane < 6))
        ceq_q = ssum(lane == 6 + q)
        eq_before = ssum((lane >= 6) & (lane < 6 + q))
        eq_take = jnp.clip(m_take - eq_before, 0, ceq_q)

        tv = jnp.full((16,), t, jnp.int32)
        capv = jnp.full((16,), eq_take, jnp.int32)
        wpv = jnp.full((16,), wp, jnp.int32)
        r0 = q * rq
        base = b * tot + r0 * wp
        posbase = jnp.full((16,), r0 * wp, jnp.int32) + lane

        def scan_quarter(nrows):
            nelems = nrows * wp
            pltpu.sync_copy(keys_hbm.at[pl.ds(base, nelems)],
                            kbuf.at[pl.ds(0, nelems)])

            def blk(ib, carry):
                gt_off, eq_off = carry
                for u in range(8):
                    kv = kbuf[pl.ds(ib * 128 + u * L, L)]
                    pos = posbase + ib * 128 + u * L
                    y = lax.div(pos, wpv)
                    gvec = pos - y * (wp - r)
                    mgt = kv > tv
                    pgt = plsc.cumsum(mgt.astype(jnp.int32))
                    plsc.store_scatter(gtbuf, [gt_off + pgt - 1], gvec,
                                       mask=mgt)
                    gt_off = gt_off + plsc.all_reduce_population_count(mgt)
                    meq = kv == tv
                    peq = plsc.cumsum(meq.astype(jnp.int32))
                    pose = eq_off + peq - 1
                    plsc.store_scatter(eqbuf, [pose], gvec,
                                       mask=meq & (pose < capv))
                    eq_off = eq_off + plsc.all_reduce_population_count(meq)
                return gt_off, eq_off

            z16 = jnp.zeros((16,), jnp.int32)
            lax.fori_loop(0, nelems // 128, blk, (z16, z16))

        pl.when(q < 3)(lambda: scan_quarter(rq))
        pl.when(q == 3)(lambda: scan_quarter(r - 3 * rq))

        def write_list(buf, cnt, gstart):
            nwaves = lax.div(cnt + 511, 512)

            def wave(w, _):
                cps = []
                for u in range(4):
                    cb = gstart + w * 512 + u * 128
                    for jj in range(8):
                        pos = jnp.full((16,), 0, jnp.int32) + cb + jj * L + lane
                        pos = jnp.where(pos < gstart + cnt, pos,
                                        BZ * K + lane)
                        offs_ref[u, pl.ds(jj * L, L)] = pos
                    cps.append(pltpu.async_copy(
                        buf.at[pl.ds(w * 512 + u * 128, 128)],
                        out_hbm.at[offs_ref.at[u]], sem))
                for cp in cps:
                    cp.wait()
                return 0

            lax.fori_loop(0, nwaves, wave, 0)

        write_list(gtbuf, cgt_q, b * K + gt_before)
        write_list(eqbuf, eq_take,
                   b * K + c_gt_tot + jnp.minimum(eq_before, m_take))

    def call(keys_flat, stats2):
        return pl.kernel(
            body,
            out_type=jax.ShapeDtypeStruct((BZ * K + 128,), jnp.int32),
            mesh=mesh,
            scratch_types=[
                pltpu.VMEM((rq * wp,), jnp.int32),
                pltpu.VMEM((K + 512,), jnp.int32),
                pltpu.VMEM((K + 512,), jnp.int32),
                pltpu.VMEM((16,), jnp.int32),
                pltpu.VMEM((4, 128), jnp.int32),
                pltpu.SemaphoreType.DMA,
            ],
            compiler_params=pltpu.CompilerParams(needs_layout_passes=False),
        )(keys_flat, stats2)

    return call


def _mk_scatter_kernel(r, wp, has_invalid):
    tot = r * wp
    share = tot // 4
    nck = max(1, round(share / 8192))
    while share % nck:
        nck += 1
    ch2 = share // nck
    seg = K // 4

    def body(occ_in, idx_hbm, vals_hbm, occ_out, cbuf0, cbuf1, ibuf, vbuf,
             offs_ref, sem, semr, semw):
        cid = lax.axis_index("c")
        sid = lax.axis_index("s")
        b = cid * 4 + lax.div(sid, 4)
        qq = lax.rem(sid, 4)
        soff = b * K + qq * seg
        cpi = pltpu.async_copy(idx_hbm.at[pl.ds(soff, seg)], ibuf, sem)
        cpv = pltpu.async_copy(vals_hbm.at[pl.ds(soff, seg)], vbuf, sem)

        my_off = cid * (4 * tot) + sid * share

        def cpair(i2, _):
            o0 = my_off + i2 * (2 * ch2)
            rd0 = pltpu.async_copy(occ_in.at[pl.ds(o0, ch2)], cbuf0, semr)
            rd1 = pltpu.async_copy(occ_in.at[pl.ds(o0 + ch2, ch2)], cbuf1,
                                   semr)
            rd0.wait()
            wr0 = pltpu.async_copy(cbuf0, occ_out.at[pl.ds(o0, ch2)], semw)
            rd1.wait()
            wr1 = pltpu.async_copy(cbuf1, occ_out.at[pl.ds(o0 + ch2, ch2)],
                                   semw)
            wr0.wait()
            wr1.wait()
            return 0

        lax.fori_loop(0, nck // 2, cpair, 0)
        if nck % 2:
            o0 = my_off + (nck - 1) * ch2
            pltpu.sync_copy(occ_in.at[pl.ds(o0, ch2)], cbuf0)
            pltpu.sync_copy(cbuf0, occ_out.at[pl.ds(o0, ch2)])
        plsc.subcore_barrier()

        cpi.wait()
        cpv.wait()
        lane = lax.iota(jnp.int32, 16)
        rsplat = jnp.full((16,), r, jnp.int32)
        def wave(wv, _):
            cps = []
            for u in range(4):
                cb = wv * 512 + u * 128
                for jj in range(8):
                    g = ibuf[pl.ds(cb + jj * L, L)]
                    y = lax.div(g, rsplat)
                    o = g + y * (wp - r) + b * tot
                    if has_invalid:
                        o = jnp.where(g < r * r, o, b * tot + wp - 1)
                    offs_ref[u, pl.ds(jj * L, L)] = o
                cps.append(pltpu.async_copy(vbuf.at[pl.ds(cb, 128)],
                                            occ_out.at[offs_ref.at[u]], sem))
            for cp in cps:
                cp.wait()
            return 0

        lax.fori_loop(0, seg // 512, wave, 0)

    mesh = plsc.VectorSubcoreMesh(core_axis_name="c", subcore_axis_name="s")

    def call(occ_in_flat, idx_flat, vals_flat):
        return pl.kernel(
            body,
            out_type=jax.ShapeDtypeStruct((BZ * tot,), jnp.float32),
            mesh=mesh,
            scratch_types=[
                pltpu.VMEM((ch2,), jnp.float32),
                pltpu.VMEM((ch2,), jnp.float32),
                pltpu.VMEM((seg,), jnp.int32),
                pltpu.VMEM((seg,), jnp.float32),
                pltpu.VMEM((4, 128), jnp.int32),
                pltpu.SemaphoreType.DMA,
                pltpu.SemaphoreType.DMA,
                pltpu.SemaphoreType.DMA,
            ],
            compiler_params=pltpu.CompilerParams(needs_layout_passes=False),
        )(occ_in_flat, idx_flat, vals_flat)

    return call


def kernel(feat, W1, Wf, b1, W2, b2, W3, b3):
    b1r = b1.reshape(1, HID)
    b2r = b2.reshape(1, HID)
    w3r = W3.reshape(1, HID)
    b3r = b3.reshape(1, 1)

    r0, wp0 = LEVELS[0]
    idx65 = jnp.broadcast_to(jnp.arange(K, dtype=jnp.int32)[None], (BZ, K))
    vals = _mk_mlp_kernel(r0, 512.0 / (r0 - 1))(
        idx65.reshape(BZ, K, 1), feat, W1, Wf, b1r, W2, b2r, w3r, b3r)
    occ_flat = _mk_scatter_kernel(r0, wp0, True)(
        jnp.zeros((BZ * r0 * wp0,), jnp.float32),
        idx65.reshape(BZ * K),
        lax.optimization_barrier(vals.reshape(BZ * K)))
    occ = occ_flat.reshape(BZ, r0, wp0)

    rp, wpp = r0, wp0
    for r, wp in LEVELS[1:]:
        m, mt = _up_mats(rp, wpp, r, wp)
        occ_up, keys, stats = _mk_up_kernel(rp, wpp, r, wp)(occ, m, mt)
        idx_pad = _mk_compact_kernel(r, wp)(
            lax.optimization_barrier(keys.reshape(BZ * r * wp)),
            lax.optimization_barrier(stats.reshape(BZ, 16)))
        idx = lax.optimization_barrier(idx_pad[:BZ * K])
        vals = _mk_mlp_kernel(r, 512.0 / (r - 1))(
            idx.reshape(BZ, K, 1), feat, W1, Wf, b1r, W2, b2r, w3r, b3r)
        occ_flat = _mk_scatter_kernel(r, wp, False)(
            lax.optimization_barrier(occ_up.reshape(BZ * r * wp)), idx,
            lax.optimization_barrier(vals.reshape(BZ * K)))
        occ = occ_flat.reshape(BZ, r, wp)
        rp, wpp = r, wp

    return occ[:, :rp, :rp].reshape(BZ, 1, rp, rp)


# speedup vs baseline: 1.0227x; 1.0227x over previous
import functools

import jax
import jax.numpy as jnp
import numpy as np
from jax import lax
from jax.experimental import pallas as pl
from jax.experimental.pallas import tpu as pltpu
from jax.experimental.pallas import tpu_sc as plsc

BZ = 8
HID = 256
K = 8192
RES_LAST = 513
KEY_MAX = 0x3F000000
NC, NS, L = 2, 16, 16

LEVELS = [(65, 128), (129, 256), (257, 384), (513, 640)]


def _up_mats(rp, wpp, r, wp):
    m = np.zeros((r, rp), np.float32)
    i = np.arange(rp)
    m[2 * i, i] = 1.0
    i = np.arange(rp - 1)
    m[2 * i + 1, i] = 0.5
    m[2 * i + 1, i + 1] = 0.5
    mt = np.zeros((wpp, wp), np.float32)
    mt[:rp, :r] = m.T
    return jnp.asarray(m), jnp.asarray(mt)


def _mk_up_kernel(rp, wpp, r, wp):
    rq = -(-r // 4)

    def body(prev_ref, m_ref, mt_ref, occ_ref, keys_ref, stats_ref):
        prev = prev_ref[0]
        tmp = jnp.dot(m_ref[:, :], prev, preferred_element_type=jnp.float32)
        out = jnp.dot(tmp, mt_ref[:, :], preferred_element_type=jnp.float32)
        occ_ref[0] = out
        a = jnp.abs(out - 0.5)
        key = KEY_MAX - lax.bitcast_convert_type(a, jnp.int32)
        col = lax.broadcasted_iota(jnp.int32, (r, wp), 1)
        key = jnp.where(col < r, key, -1)
        keys_ref[0] = key

        def bs(_, lohi):
            lo, hi = lohi
            mid = lo + (hi - lo + 1) // 2
            cnt = jnp.sum((key >= mid).astype(jnp.int32))
            ge = cnt >= K
            return (jnp.where(ge, mid, lo), jnp.where(ge, hi, mid - 1))

        lo, _ = lax.fori_loop(
            0, 31, bs, (jnp.int32(0), jnp.int32(KEY_MAX + 1)))
        t = lo
        mgt = (key > t).astype(jnp.int32)
        meq = (key == t).astype(jnp.int32)
        cg = [jnp.sum(mgt[q * rq:min((q + 1) * rq, r)]) for q in range(4)]
        ce = [jnp.sum(meq[q * rq:min((q + 1) * rq, r)]) for q in range(4)]
        c_gt = cg[0] + cg[1] + cg[2] + cg[3]
        m_take = K - c_gt
        vals = [t, m_take] + cg + ce + [jnp.int32(0)] * 6
        ii = lax.broadcasted_iota(jnp.int32, (1, 16), 1)
        acc = jnp.zeros((1, 16), jnp.int32)
        for e, v in enumerate(vals):
            acc = jnp.where(ii == e, v, acc)
        stats_ref[0] = acc

    def call(occ_prev, m, mt):
        return pl.pallas_call(
            body,
            grid=(BZ,),
            in_specs=[
                pl.BlockSpec((1, rp, wpp), lambda b: (b, 0, 0)),
                pl.BlockSpec((r, rp), lambda b: (0, 0)),
                pl.BlockSpec((wpp, wp), lambda b: (0, 0)),
            ],
            out_specs=[
                pl.BlockSpec((1, r, wp), lambda b: (b, 0, 0)),
                pl.BlockSpec((1, r, wp), lambda b: (b, 0, 0)),
                pl.BlockSpec((1, 1, 16), lambda b: (b, 0, 0)),
            ],
            out_shape=[
                jax.ShapeDtypeStruct((BZ, r, wp), jnp.float32),
                jax.ShapeDtypeStruct((BZ, r, wp), jnp.int32),
                jax.ShapeDtypeStruct((BZ, 1, 16), jnp.int32),
            ],
        )(occ_prev, m, mt)

    return call


def _mk_mlp_kernel(r, stride, wp):
    half_step = float(np.float32(np.float32(1.0 / RES_LAST) / 2))
    tot = r * wp

    def body(idx_ref, feat_ref, w1_ref, wf_ref, b1_ref, w2_ref, b2_ref,
             w3_ref, b3_ref, out_ref, offs_ref):
        idx = idx_ref[0]
        b = pl.program_id(0)
        px_i = idx % r
        py_i = idx // r
        o = jnp.where(idx < r * r, py_i * wp + px_i + b * tot,
                      b * tot + wp - 1)
        offs_ref[0] = o
        px = px_i.astype(jnp.float32) * stride
        py = py_i.astype(jnp.float32) * stride
        cx = (px / 513.0 + half_step) * 2.0 - 1.0
        cy = (py / 513.0 + half_step) * 2.0 - 1.0
        fw = jnp.dot(feat_ref[0], wf_ref[:, :],
                     preferred_element_type=jnp.float32) + b1_ref[:, :]
        h1 = jnp.maximum(cx * w1_ref[0:1, :] + cy * w1_ref[1:2, :] + fw, 0.0)
        h2 = jnp.maximum(
            jnp.dot(h1, w2_ref[:, :], preferred_element_type=jnp.float32)
            + b2_ref[:, :], 0.0)
        z = jnp.sum(h2 * w3_ref[:, :], axis=1, keepdims=True) + b3_ref[:, :]
        out_ref[0] = jax.nn.sigmoid(z)

    TK = 2048

    def call(idx3, feat, w1, wf, b1r, w2, b2r, w3r, b3r):
        return pl.pallas_call(
            body,
            grid=(BZ, K // TK),
            in_specs=[
                pl.BlockSpec((1, TK, 1), lambda b, t: (b, t, 0)),
                pl.BlockSpec((1, 1, HID), lambda b, t: (b, 0, 0)),
                pl.BlockSpec((2, HID), lambda b, t: (0, 0)),
                pl.BlockSpec((HID, HID), lambda b, t: (0, 0)),
                pl.BlockSpec((1, HID), lambda b, t: (0, 0)),
                pl.BlockSpec((HID, HID), lambda b, t: (0, 0)),
                pl.BlockSpec((1, HID), lambda b, t: (0, 0)),
                pl.BlockSpec((1, HID), lambda b, t: (0, 0)),
                pl.BlockSpec((1, 1), lambda b, t: (0, 0)),
            ],
            out_specs=[
                pl.BlockSpec((1, TK, 1), lambda b, t: (b, t, 0)),
                pl.BlockSpec((1, TK, 1), lambda b, t: (b, t, 0)),
            ],
            out_shape=[
                jax.ShapeDtypeStruct((BZ, K, 1), jnp.float32),
                jax.ShapeDtypeStruct((BZ, K, 1), jnp.int32),
            ],
        )(idx3, feat.reshape(BZ, 1, HID), w1, wf, b1r, w2, b2r, w3r, b3r)

    return call


def _mk_compact_kernel(r, wp):
    rq = -(-r // 4)
    tot = r * wp
    mesh = plsc.VectorSubcoreMesh(core_axis_name="c", subcore_axis_name="s")

    def body(keys_hbm, stats_hbm, out_hbm, kbuf, gtbuf, eqbuf, statv,
             offs_ref, sem):
        cid = lax.axis_index("c")
        sid = lax.axis_index("s")
        wid = cid * NS + sid
        b = lax.div(wid, 4)
        q = lax.rem(wid, 4)
        lane = lax.iota(jnp.int32, 16)

        pltpu.sync_copy(stats_hbm.at[b], statv)
        sv = statv[...]

        def ssum(msk):
            return jnp.sum(jnp.where(msk, sv, jnp.zeros_like(sv)))

        t = ssum(lane == 0)
        m_take = ssum(lane == 1)
        cgt_q = ssum(lane == 2 + q)
        gt_before = ssum((lane >= 2) & (lane < 2 + q))
        c_gt_tot = ssum((lane >= 2) & (lane < 6))
        ceq_q = ssum(lane == 6 + q)
        eq_before = ssum((lane >= 6) & (lane < 6 + q))
        eq_take = jnp.clip(m_take - eq_before, 0, ceq_q)

        tv = jnp.full((16,), t, jnp.int32)
        capv = jnp.full((16,), eq_take, jnp.int32)
        r0 = q * rq
        base = b * tot + r0 * wp

        def scan_quarter(nrows):
            nelems = nrows * wp
            pltpu.sync_copy(keys_hbm.at[pl.ds(base, nelems)],
                            kbuf.at[pl.ds(0, nelems)])

            def row(r2, carry):
                gt_off, eq_off = carry
                gb = jnp.full((16,), (r0 + r2) * r, jnp.int32) + lane
                ko = r2 * wp
                for u in range(wp // L):
                    kv = kbuf[pl.ds(ko + u * L, L)]
                    gvec = gb + u * L
                    mgt = kv > tv
                    pgt = plsc.cumsum(mgt.astype(jnp.int32))
                    plsc.store_scatter(gtbuf, [gt_off + pgt - 1], gvec,
                                       mask=mgt)
                    gt_off = gt_off + plsc.all_reduce_population_count(mgt)
                    meq = kv == tv
                    peq = plsc.cumsum(meq.astype(jnp.int32))
                    pose = eq_off + peq - 1
                    plsc.store_scatter(eqbuf, [pose], gvec,
                                       mask=meq & (pose < capv))
                    eq_off = eq_off + plsc.all_reduce_population_count(meq)
                return gt_off, eq_off

            z16 = jnp.zeros((16,), jnp.int32)
            lax.fori_loop(0, nrows, row, (z16, z16))

        pl.when(q < 3)(lambda: scan_quarter(rq))
        pl.when(q == 3)(lambda: scan_quarter(r - 3 * rq))

        def write_list(buf, cnt, gstart):
            nwaves = lax.div(cnt + 511, 512)

            def wave(w, _):
                cps = []
                for u in range(4):
                    cb = gstart + w * 512 + u * 128
                    for jj in range(8):
                        pos = jnp.full((16,), 0, jnp.int32) + cb + jj * L + lane
                        pos = jnp.where(pos < gstart + cnt, pos,
                                        BZ * K + lane)
                        offs_ref[u, pl.ds(jj * L, L)] = pos
                    cps.append(pltpu.async_copy(
                        buf.at[pl.ds(w * 512 + u * 128, 128)],
                        out_hbm.at[offs_ref.at[u]], sem))
                for cp in cps:
                    cp.wait()
                return 0

            lax.fori_loop(0, nwaves, wave, 0)

        write_list(gtbuf, cgt_q, b * K + gt_before)
        write_list(eqbuf, eq_take,
                   b * K + c_gt_tot + jnp.minimum(eq_before, m_take))

    def call(keys_flat, stats2):
        return pl.kernel(
            body,
            out_type=jax.ShapeDtypeStruct((BZ * K + 128,), jnp.int32),
            mesh=mesh,
            scratch_types=[
                pltpu.VMEM((rq * wp,), jnp.int32),
                pltpu.VMEM((K + 512,), jnp.int32),
                pltpu.VMEM((K + 512,), jnp.int32),
                pltpu.VMEM((16,), jnp.int32),
                pltpu.VMEM((4, 128), jnp.int32),
                pltpu.SemaphoreType.DMA,
            ],
            compiler_params=pltpu.CompilerParams(needs_layout_passes=False),
        )(keys_flat, stats2)

    return call


def _mk_scatter_kernel(r, wp, has_invalid):
    tot = r * wp
    share = tot // 4
    nck = max(1, round(share / 8192))
    while share % nck:
        nck += 1
    ch2 = share // nck
    seg = K // 4

    def body(occ_in, offs_hbm, vals_hbm, occ_out, cbuf0, cbuf1, vbuf,
             offs_ref, sem, semr, semw):
        cid = lax.axis_index("c")
        sid = lax.axis_index("s")
        b = cid * 4 + lax.div(sid, 4)
        qq = lax.rem(sid, 4)
        soff = b * K + qq * seg
        cpi = pltpu.async_copy(offs_hbm.at[b, pl.ds(qq * (seg // 128),
                                                    seg // 128)],
                               offs_ref, sem)
        cpv = pltpu.async_copy(vals_hbm.at[pl.ds(soff, seg)], vbuf, sem)

        my_off = cid * (4 * tot) + sid * share

        def cpair(i2, _):
            o0 = my_off + i2 * (2 * ch2)
            rd0 = pltpu.async_copy(occ_in.at[pl.ds(o0, ch2)], cbuf0, semr)
            rd1 = pltpu.async_copy(occ_in.at[pl.ds(o0 + ch2, ch2)], cbuf1,
                                   semr)
            rd0.wait()
            wr0 = pltpu.async_copy(cbuf0, occ_out.at[pl.ds(o0, ch2)], semw)
            rd1.wait()
            wr1 = pltpu.async_copy(cbuf1, occ_out.at[pl.ds(o0 + ch2, ch2)],
                                   semw)
            wr0.wait()
            wr1.wait()
            return 0

        lax.fori_loop(0, nck // 2, cpair, 0)
        if nck % 2:
            o0 = my_off + (nck - 1) * ch2
            pltpu.sync_copy(occ_in.at[pl.ds(o0, ch2)], cbuf0)
            pltpu.sync_copy(cbuf0, occ_out.at[pl.ds(o0, ch2)])
        plsc.subcore_barrier()

        cpi.wait()
        cpv.wait()
        cps = []
        for ci in range(seg // 128):
            cps.append(pltpu.async_copy(vbuf.at[pl.ds(ci * 128, 128)],
                                        occ_out.at[offs_ref.at[ci]], sem))
        for cp in cps:
            cp.wait()

    mesh = plsc.VectorSubcoreMesh(core_axis_name="c", subcore_axis_name="s")

    def call(occ_in_flat, offs3, vals_flat):
        return pl.kernel(
            body,
            out_type=jax.ShapeDtypeStruct((BZ * tot,), jnp.float32),
            mesh=mesh,
            scratch_types=[
                pltpu.VMEM((ch2,), jnp.float32),
                pltpu.VMEM((ch2,), jnp.float32),
                pltpu.VMEM((seg,), jnp.float32),
                pltpu.VMEM((seg // 128, 128), jnp.int32),
                pltpu.SemaphoreType.DMA,
                pltpu.SemaphoreType.DMA,
                pltpu.SemaphoreType.DMA,
            ],
            compiler_params=pltpu.CompilerParams(needs_layout_passes=False),
        )(occ_in_flat, offs3, vals_flat)

    return call


def kernel(feat, W1, Wf, b1, W2, b2, W3, b3):
    b1r = b1.reshape(1, HID)
    b2r = b2.reshape(1, HID)
    w3r = W3.reshape(1, HID)
    b3r = b3.reshape(1, 1)

    r0, wp0 = LEVELS[0]
    idx65 = jnp.broadcast_to(jnp.arange(K, dtype=jnp.int32)[None], (BZ, K))
    vals, offs = _mk_mlp_kernel(r0, 512.0 / (r0 - 1), wp0)(
        idx65.reshape(BZ, K, 1), feat, W1, Wf, b1r, W2, b2r, w3r, b3r)
    occ_flat = _mk_scatter_kernel(r0, wp0, True)(
        jnp.zeros((BZ * r0 * wp0,), jnp.float32),
        lax.optimization_barrier(offs.reshape(BZ, K // 128, 128)),
        lax.optimization_barrier(vals.reshape(BZ * K)))
    occ = occ_flat.reshape(BZ, r0, wp0)

    rp, wpp = r0, wp0
    for r, wp in LEVELS[1:]:
        m, mt = _up_mats(rp, wpp, r, wp)
        occ_up, keys, stats = _mk_up_kernel(rp, wpp, r, wp)(occ, m, mt)
        idx_pad = _mk_compact_kernel(r, wp)(
            lax.optimization_barrier(keys.reshape(BZ * r * wp)),
            lax.optimization_barrier(stats.reshape(BZ, 16)))
        idx = lax.optimization_barrier(idx_pad[:BZ * K])
        vals, offs = _mk_mlp_kernel(r, 512.0 / (r - 1), wp)(
            idx.reshape(BZ, K, 1), feat, W1, Wf, b1r, W2, b2r, w3r, b3r)
        occ_flat = _mk_scatter_kernel(r, wp, False)(
            lax.optimization_barrier(occ_up.reshape(BZ * r * wp)),
            lax.optimization_barrier(offs.reshape(BZ, K // 128, 128)),
            lax.optimization_barrier(vals.reshape(BZ * K)))
        occ = occ_flat.reshape(BZ, r, wp)
        rp, wpp = r, wp

    return occ[:, :rp, :rp].reshape(BZ, 1, rp, rp)
